# Initial kernel scaffold; baseline (speedup 1.0000x reference)
import jax
import jax.numpy as jnp
from jax.experimental import pallas as pl


def kernel(inputs, inputs_up, inputs_down, targets, epoch, features, features_up, features_down):
    raise NotImplementedError("write your pallas kernel here")


# speedup vs baseline: 4.1429x; 4.1429x over previous
import jax
import jax.numpy as jnp
from jax.experimental import pallas as pl
from jax.experimental.pallas import tpu as pltpu

_TEMP = 0.05
_L2 = 0.5
_B = 1024
_F = 128
_N = 100000
_C = 1000
_STEPS = _N // _C
_SHIFT = 1.0 / _TEMP


def _loss_body(x_ref, xu_ref, xd_ref, t_ref, f_ref, fu_ref, fd_ref,
               out_ref, xs, se, tl):
    c = pl.program_id(0)

    @pl.when(c == 0)
    def _init():
        for k, r in enumerate((x_ref, xu_ref, xd_ref)):
            v = r[...]
            n = jnp.sqrt(jnp.sum(v * v, axis=1, keepdims=True))
            xs[k] = v / jnp.maximum(n, 1e-12)
        se[...] = jnp.zeros_like(se)
        tl[...] = jnp.zeros_like(tl)

    ids = jax.lax.broadcasted_iota(jnp.int32, (_B, _C), 1) + c * _C
    mask = ids == t_ref[...]
    for k, fr in enumerate((f_ref, fu_ref, fd_ref)):
        logits = jax.lax.dot_general(
            xs[k], fr[...], (((1,), (1,)), ((), ())),
            preferred_element_type=jnp.float32) * (1.0 / _TEMP)
        se[k] += jnp.sum(jnp.exp(logits - _SHIFT), axis=1, keepdims=True)
        tl[k] += jnp.sum(jnp.where(mask, logits, 0.0), axis=1, keepdims=True)

    @pl.when(c == _STEPS - 1)
    def _fin():
        acc = jnp.float32(0.0)
        for k, w in enumerate((1.0 - _L2, _L2, _L2)):
            nll = jnp.log(se[k]) + _SHIFT - tl[k]
            acc += w * jnp.sum(nll)
        out_ref[...] = (acc / _B).reshape(1, 1)


def _fused_loss(x, xu, xd, t, f, fu, fd):
    full = pl.BlockSpec((_B, _F), lambda c: (0, 0))
    return pl.pallas_call(
        _loss_body,
        grid=(_STEPS,),
        in_specs=[
            full, full, full,
            pl.BlockSpec((_B, 1), lambda c: (0, 0)),
            pl.BlockSpec((_C, _F), lambda c: (c, 0)),
            pl.BlockSpec((_C, _F), lambda c: (c, 0)),
            pl.BlockSpec((_C, _F), lambda c: (c, 0)),
        ],
        out_specs=pl.BlockSpec((1, 1), lambda c: (0, 0)),
        out_shape=jax.ShapeDtypeStruct((1, 1), jnp.float32),
        scratch_shapes=[
            pltpu.VMEM((3, _B, _F), jnp.float32),
            pltpu.VMEM((3, _B, 1), jnp.float32),
            pltpu.VMEM((3, _B, 1), jnp.float32),
        ],
    )(x, xu, xd, t, f, fu, fd)


def kernel(inputs, inputs_up, inputs_down, targets, epoch,
           features, features_up, features_down):
    del epoch
    t2 = targets.reshape(_B, 1)
    loss = _fused_loss(inputs, inputs_up, inputs_down, t2,
                       features, features_up, features_down)
    return loss[0, 0]


# speedup vs baseline: 5.1332x; 1.2390x over previous
import jax
import jax.numpy as jnp
from jax.experimental import pallas as pl
from jax.experimental.pallas import tpu as pltpu

_TEMP = 0.05
_L2 = 0.5
_B = 1024
_F = 128
_N = 100000
_C = 1000
_STEPS = _N // _C
_LOG2E = 1.4426950408889634
_SHIFT2 = _LOG2E / _TEMP
_LN2 = 0.6931471805599453


def _loss_body(x_ref, xu_ref, xd_ref, t_ref, f_ref, fu_ref, fd_ref,
               out_ref, xs, se, tl):
    c = pl.program_id(0)

    @pl.when(c == 0)
    def _init():
        for k, r in enumerate((x_ref, xu_ref, xd_ref)):
            v = r[...]
            n = jnp.sqrt(jnp.sum(v * v, axis=1, keepdims=True))
            xs[k] = v * (_SHIFT2 / jnp.maximum(n, 1e-12))
        se[...] = jnp.zeros_like(se)
        tl[...] = jnp.zeros_like(tl)

    ids = jax.lax.broadcasted_iota(jnp.int32, (_B, _C), 1) + c * _C
    mask = ids == t_ref[...]
    for k, fr in enumerate((f_ref, fu_ref, fd_ref)):
        y = jax.lax.dot_general(
            xs[k], fr[...], (((1,), (1,)), ((), ())),
            preferred_element_type=jnp.float32)
        se[k] += jnp.sum(jnp.exp2(y - _SHIFT2), axis=1, keepdims=True)
        tl[k] += jnp.sum(jnp.where(mask, y, 0.0), axis=1, keepdims=True)

    @pl.when(c == _STEPS - 1)
    def _fin():
        acc = jnp.float32(0.0)
        for k, w in enumerate((1.0 - _L2, _L2, _L2)):
            nll = _LN2 * (jnp.log2(se[k]) + _SHIFT2 - tl[k])
            acc += w * jnp.sum(nll)
        out_ref[...] = (acc / _B).reshape(1, 1)


def _fused_loss(x, xu, xd, t, f, fu, fd):
    full = pl.BlockSpec((_B, _F), lambda c: (0, 0))
    return pl.pallas_call(
        _loss_body,
        grid=(_STEPS,),
        in_specs=[
            full, full, full,
            pl.BlockSpec((_B, 1), lambda c: (0, 0)),
            pl.BlockSpec((_C, _F), lambda c: (c, 0)),
            pl.BlockSpec((_C, _F), lambda c: (c, 0)),
            pl.BlockSpec((_C, _F), lambda c: (c, 0)),
        ],
        out_specs=pl.BlockSpec((1, 1), lambda c: (0, 0)),
        out_shape=jax.ShapeDtypeStruct((1, 1), jnp.float32),
        scratch_shapes=[
            pltpu.VMEM((3, _B, _F), jnp.float32),
            pltpu.VMEM((3, _B, 1), jnp.float32),
            pltpu.VMEM((3, _B, 1), jnp.float32),
        ],
    )(x, xu, xd, t, f, fu, fd)


def kernel(inputs, inputs_up, inputs_down, targets, epoch,
           features, features_up, features_down):
    del epoch
    t2 = targets.reshape(_B, 1)
    loss = _fused_loss(inputs, inputs_up, inputs_down, t2,
                       features, features_up, features_down)
    return loss[0, 0]


# speedup vs baseline: 6.9110x; 1.3463x over previous
import functools

import jax
import jax.numpy as jnp
from jax import lax
from jax.experimental import pallas as pl
from jax.experimental.pallas import tpu as pltpu
from jax.experimental.pallas import tpu_sc as plsc

_TEMP = 0.05
_L2 = 0.5
_B = 1024
_F = 128
_N = 100000
_C = 1000
_STEPS = _N // _C
_LOG2E = 1.4426950408889634
_SHIFT2 = _LOG2E / _TEMP
_LN2 = 0.6931471805599453


def _gather_targets(f, fu, fd, targets):
    info = plsc.get_sparse_core_info()
    nw = info.num_cores * info.num_subcores
    bpw = _B // nw
    mesh = plsc.VectorSubcoreMesh(core_axis_name="c", subcore_axis_name="s")

    @functools.partial(
        pl.kernel, mesh=mesh,
        out_type=[jax.ShapeDtypeStruct((_B, _F), jnp.float32)] * 3,
        scratch_types=[
            pltpu.VMEM((bpw,), jnp.int32),
            pltpu.VMEM((bpw, _F), jnp.float32),
            pltpu.SemaphoreType.DMA,
        ],
    )
    def gather3(t_hbm, f0, f1, f2, o0, o1, o2, idx_v, rows_v, sem):
        wid = lax.axis_index("s") * info.num_cores + lax.axis_index("c")
        base = wid * bpw
        pltpu.sync_copy(t_hbm.at[pl.ds(base, bpw)], idx_v)
        for t, o in ((f0, o0), (f1, o1), (f2, o2)):
            pltpu.async_copy(t.at[idx_v], rows_v, sem).wait()
            pltpu.sync_copy(rows_v, o.at[pl.ds(base, bpw)])

    return gather3(targets, f, fu, fd)


def _loss_body(x_ref, xu_ref, xd_ref, g_ref, gu_ref, gd_ref,
               f_ref, fu_ref, fd_ref, out_ref, xs, se):
    c = pl.program_id(0)

    @pl.when(c == 0)
    def _init():
        for k, r in enumerate((x_ref, xu_ref, xd_ref)):
            v = r[...]
            n = jnp.sqrt(jnp.sum(v * v, axis=1, keepdims=True))
            xs[k] = v * (_SHIFT2 / jnp.maximum(n, 1e-12))
        se[...] = jnp.zeros_like(se)

    for k, fr in enumerate((f_ref, fu_ref, fd_ref)):
        y = jax.lax.dot_general(
            xs[k], fr[...], (((1,), (1,)), ((), ())),
            preferred_element_type=jnp.float32)
        se[k] += jnp.sum(jnp.exp2(y - _SHIFT2), axis=1, keepdims=True)

    @pl.when(c == _STEPS - 1)
    def _fin():
        acc = jnp.float32(0.0)
        for k, (w, gr) in enumerate(zip((1.0 - _L2, _L2, _L2),
                                        (g_ref, gu_ref, gd_ref))):
            yt = jnp.sum(xs[k] * gr[...], axis=1, keepdims=True)
            nll = _LN2 * (jnp.log2(se[k]) + _SHIFT2 - yt)
            acc += w * jnp.sum(nll)
        out_ref[...] = (acc / _B).reshape(1, 1)


def _fused_loss(x, xu, xd, g, gu, gd, f, fu, fd):
    full = pl.BlockSpec((_B, _F), lambda c: (0, 0))
    return pl.pallas_call(
        _loss_body,
        grid=(_STEPS,),
        in_specs=[
            full, full, full, full, full, full,
            pl.BlockSpec((_C, _F), lambda c: (c, 0)),
            pl.BlockSpec((_C, _F), lambda c: (c, 0)),
            pl.BlockSpec((_C, _F), lambda c: (c, 0)),
        ],
        out_specs=pl.BlockSpec((1, 1), lambda c: (0, 0)),
        out_shape=jax.ShapeDtypeStruct((1, 1), jnp.float32),
        scratch_shapes=[
            pltpu.VMEM((3, _B, _F), jnp.float32),
            pltpu.VMEM((3, _B, 1), jnp.float32),
        ],
    )(x, xu, xd, g, gu, gd, f, fu, fd)


def kernel(inputs, inputs_up, inputs_down, targets, epoch,
           features, features_up, features_down):
    del epoch
    g, gu, gd = _gather_targets(features, features_up, features_down, targets)
    loss = _fused_loss(inputs, inputs_up, inputs_down, g, gu, gd,
                       features, features_up, features_down)
    return loss[0, 0]
